# baseline (speedup 1.0000x reference)
import functools

import jax
import jax.numpy as jnp
from jax import lax
from jax.experimental import pallas as pl
from jax.experimental.pallas import tpu as pltpu
from jax.experimental.pallas import tpu_sc as plsc

B = 16384
L = 20
NC = 2
NS = 16
NW = NC * NS
BPW = B // NW
CH = 128

ID_CHUNKS = BPW // CH
SMALL_ROWS = 5 * B
SMALL_IDX_PW = SMALL_ROWS // NW // CH
LAB_ROWS = 2 * B * L
LAB_IDX_PW = LAB_ROWS // NW // CH
LAB_INNER = 16
LAB_OUTER = LAB_IDX_PW // LAB_INNER
SMALL_INNER = 10
SMALL_OUTER = SMALL_IDX_PW // SMALL_INNER
EG_ROWS_PW = BPW * 64 // CH
EG_INNER = 16
EG_OUTER = EG_ROWS_PW // EG_INNER


def _mesh():
  return plsc.VectorSubcoreMesh(core_axis_name="c", subcore_axis_name="s",
                                num_cores=NC, num_subcores=NS)


def _wid():
  return lax.axis_index("s") * NC + lax.axis_index("c")


def _sc_gather_body(uid_idx, iid_idx, cat_idx, small_idx, lab_idx,
                    uid_tbl, iid_tbl, cat_tbl, small_tbl, lab_tbl,
                    uid_out, iid_out, cat_out, small_out, lab_out,
                    idxv, erows, rows32, rowsc, rows16, sem):
  wid = _wid()

  def rows_gather(idx_hbm, tbl, out_hbm, rowsv):
    pltpu.sync_copy(idx_hbm.at[wid], idxv.at[pl.ds(0, ID_CHUNKS)])
    descs = []
    for j in range(ID_CHUNKS):
      descs.append(
          pltpu.async_copy(tbl.at[idxv.at[j]],
                           rowsv.at[pl.ds(j * CH, CH)], sem))
    for d in descs:
      d.wait()
    pltpu.sync_copy(rowsv, out_hbm.at[pl.ds(wid * ID_CHUNKS * CH,
                                            ID_CHUNKS * CH)])

  def elem_gather(eidx_hbm, tbl1d, out_hbm):
    def echunk(c, carry):
      pltpu.sync_copy(eidx_hbm.at[wid, pl.ds(c * EG_INNER, EG_INNER)],
                      idxv.at[pl.ds(0, EG_INNER)])
      descs = []
      for j in range(EG_INNER):
        descs.append(
            pltpu.async_copy(tbl1d.at[idxv.at[j]], erows.at[j], sem))
      for d in descs:
        d.wait()
      pltpu.sync_copy(
          erows,
          out_hbm.at[pl.ds(wid * EG_ROWS_PW + c * EG_INNER, EG_INNER)])
      return carry

    lax.fori_loop(0, EG_OUTER, echunk, 0)

  elem_gather(uid_idx, uid_tbl, uid_out)
  elem_gather(iid_idx, iid_tbl, iid_out)
  rows_gather(cat_idx, cat_tbl, cat_out, rowsc)

  pltpu.sync_copy(small_idx.at[wid], idxv.at[pl.ds(0, SMALL_IDX_PW)])

  def small_chunk(c, carry):
    descs = []
    for j in range(SMALL_INNER):
      descs.append(
          pltpu.async_copy(small_tbl.at[idxv.at[c * SMALL_INNER + j]],
                           rows16.at[pl.ds(j * CH, CH)], sem))
    for d in descs:
      d.wait()
    pltpu.sync_copy(
        rows16,
        small_out.at[pl.ds(wid * SMALL_IDX_PW * CH + c * SMALL_INNER * CH,
                           SMALL_INNER * CH)])
    return carry

  lax.fori_loop(0, SMALL_OUTER, small_chunk, 0)

  def lab_chunk(c, carry):
    pltpu.sync_copy(lab_idx.at[wid, pl.ds(c * LAB_INNER, LAB_INNER)],
                    idxv.at[pl.ds(0, LAB_INNER)])
    descs = []
    for j in range(LAB_INNER):
      descs.append(
          pltpu.async_copy(lab_tbl.at[idxv.at[j]],
                           rows32.at[pl.ds(j * CH, CH)], sem))
    for d in descs:
      d.wait()
    pltpu.sync_copy(
        rows32,
        lab_out.at[pl.ds(wid * LAB_IDX_PW * CH + c * LAB_INNER * CH,
                         LAB_INNER * CH)])
    return carry

  lax.fori_loop(0, LAB_OUTER, lab_chunk, 0)


def _sc_gather(uid_idx, iid_idx, cat_idx, small_idx, lab_idx,
               uid_tbl, iid_tbl, cat_tbl, small_tbl, lab_tbl):
  f = pl.kernel(
      _sc_gather_body,
      out_type=(
          jax.ShapeDtypeStruct((B * 64 // CH, CH), jnp.float32),
          jax.ShapeDtypeStruct((B * 64 // CH, CH), jnp.float32),
          jax.ShapeDtypeStruct((B, 32), jnp.float32),
          jax.ShapeDtypeStruct((SMALL_ROWS, 16), jnp.float32),
          jax.ShapeDtypeStruct((LAB_ROWS, 32), jnp.bfloat16),
      ),
      mesh=_mesh(),
      compiler_params=pltpu.CompilerParams(use_tc_tiling_on_sc=False),
      scratch_types=[
          pltpu.VMEM((SMALL_IDX_PW, CH), jnp.int32),
          pltpu.VMEM((EG_INNER, CH), jnp.float32),
          pltpu.VMEM((LAB_INNER * CH, 32), jnp.bfloat16),
          pltpu.VMEM((ID_CHUNKS * CH, 32), jnp.float32),
          pltpu.VMEM((SMALL_INNER * CH, 16), jnp.float32),
          pltpu.SemaphoreType.DMA,
      ],
  )
  return f(uid_idx, iid_idx, cat_idx, small_idx, lab_idx,
           uid_tbl, iid_tbl, cat_tbl, small_tbl, lab_tbl)


BS = 512


def _tc_dense_body(uid_ref, iid_ref, cat_ref, small_ref,
                   lab_ref, w20_ref, e20_ref, p32_ref,
                   u1_ref, ub1_ref, u2_ref, ub2_ref,
                   i1_ref, ib1_ref, i2_ref, ib2_ref, out_ref):
  uid_emb = uid_ref[...]
  iid_emb = iid_ref[...]

  w20 = w20_ref[...]
  e20 = e20_ref[...]
  p32 = p32_ref[...]

  def pool(x16):
    x = x16.astype(jnp.float32)
    s = jnp.dot(x, w20, preferred_element_type=jnp.float32)
    m = jnp.max(s, axis=1, keepdims=True)
    e = jnp.exp(s - m)
    wt = e / jnp.sum(e, axis=1, keepdims=True)
    wt640 = jnp.dot(wt, e20, preferred_element_type=jnp.float32)
    return jnp.dot(x * wt640, p32, preferred_element_type=jnp.float32)

  u_pool = pool(lab_ref[0])
  i_pool = pool(lab_ref[1])

  user_feat = jnp.concatenate(
      [uid_emb, small_ref[0], small_ref[1], small_ref[2], small_ref[3],
       u_pool], axis=1)
  item_feat = jnp.concatenate(
      [iid_emb, cat_ref[...], small_ref[4], i_pool], axis=1)

  hu = jnp.maximum(
      jnp.dot(user_feat, u1_ref[...], preferred_element_type=jnp.float32)
      + ub1_ref[0], 0.0)
  uvec = jnp.dot(hu, u2_ref[...], preferred_element_type=jnp.float32) \
      + ub2_ref[0]
  hi = jnp.dot(item_feat, i1_ref[...], preferred_element_type=jnp.float32) \
      + ib1_ref[0]
  ivec = jnp.dot(hi, i2_ref[...], preferred_element_type=jnp.float32) \
      + ib2_ref[0]
  logit = jnp.sum(uvec * ivec, axis=1, keepdims=True)
  out_ref[...] = 1.0 / (1.0 + jnp.exp(-logit))


def _tc_dense(uid_emb, iid_emb, cat_emb, small_emb, lab_emb, w20, e20, p32,
              U1, Ub1, U2, Ub2, I1, Ib1, I2, Ib2):
  grid = (B // BS,)
  full = lambda shape: pl.BlockSpec(shape, lambda i: tuple(0 for _ in shape))
  out = pl.pallas_call(
      _tc_dense_body,
      grid=grid,
      in_specs=[
          pl.BlockSpec((BS, 64), lambda i: (i, 0)),
          pl.BlockSpec((BS, 64), lambda i: (i, 0)),
          pl.BlockSpec((BS, 32), lambda i: (i, 0)),
          pl.BlockSpec((5, BS, 16), lambda i: (0, i, 0)),
          pl.BlockSpec((2, BS, 640), lambda i: (0, i, 0)),
          full((640, 20)), full((20, 640)), full((640, 32)),
          full((160, 256)), full((1, 256)), full((256, 128)), full((1, 128)),
          full((144, 256)), full((1, 256)), full((256, 128)), full((1, 128)),
      ],
      out_specs=pl.BlockSpec((BS, 1), lambda i: (i, 0)),
      out_shape=jax.ShapeDtypeStruct((B, 1), jnp.float32),
  )(uid_emb, iid_emb, cat_emb, small_emb, lab_emb, w20, e20, p32,
    U1, Ub1, U2, Ub2, I1, Ib1, I2, Ib2)
  return out


def kernel(user_id, gender_id, job_id, user_city_id, age_bucket, user_labels,
           item_id, category_id, item_city_id, item_labels,
           user_id_table, gender_table, job_table, city_table, age_table,
           item_id_table, category_table, label_table, w_pool,
           U1, Ub1, U2, Ub2, I1, Ib1, I2, Ib2):
  i32 = jnp.int32
  bf16 = jnp.bfloat16
  small_tbl = jnp.concatenate(
      [gender_table, job_table, city_table, age_table], axis=0)
  small_idx = jnp.concatenate([
      gender_id.astype(i32),
      job_id.astype(i32) + 3,
      user_city_id.astype(i32) + 104,
      age_bucket.astype(i32) + 1105,
      item_city_id.astype(i32) + 104,
  ]).reshape(NW, SMALL_IDX_PW, CH)
  lab_idx = jnp.concatenate(
      [user_labels.reshape(-1).astype(i32),
       item_labels.reshape(-1).astype(i32)]).reshape(NW, LAB_IDX_PW, CH)

  dim_off = (jnp.arange(64, dtype=i32) * 1000000)[None, :]
  uid_eidx = (user_id.astype(i32)[:, None] + dim_off).reshape(NW, EG_ROWS_PW,
                                                             CH)
  iid_eidx = (item_id.astype(i32)[:, None] + dim_off).reshape(NW, EG_ROWS_PW,
                                                              CH)

  uid_emb, iid_emb, cat_emb, small_emb, lab_emb = _sc_gather(
      uid_eidx, iid_eidx,
      category_id.astype(i32).reshape(NW, ID_CHUNKS, CH),
      small_idx, lab_idx,
      user_id_table.T.reshape(64 * 1000000),
      item_id_table.T.reshape(64 * 1000000),
      category_table, small_tbl,
      label_table.astype(bf16))
  uid_emb = uid_emb.reshape(B, 64)
  iid_emb = iid_emb.reshape(B, 64)

  eye20 = jnp.eye(20, dtype=jnp.float32)
  w20 = jnp.kron(eye20, w_pool.reshape(32, 1))
  e20 = jnp.kron(eye20, jnp.ones((1, 32), jnp.float32))
  p32 = jnp.kron(jnp.ones((20, 1), jnp.float32),
                 jnp.eye(32, dtype=jnp.float32))

  out = _tc_dense(uid_emb, iid_emb, cat_emb,
                  small_emb.reshape(5, B, 16),
                  lab_emb.reshape(2, B, 640),
                  w20, e20, p32,
                  U1, Ub1.reshape(1, 256), U2, Ub2.reshape(1, 128),
                  I1, Ib1.reshape(1, 256), I2, Ib2.reshape(1, 128))
  return out.reshape(B)


# speedup vs baseline: 6.7856x; 6.7856x over previous
import functools

import jax
import jax.numpy as jnp
from jax import lax
from jax.experimental import pallas as pl
from jax.experimental.pallas import tpu as pltpu
from jax.experimental.pallas import tpu_sc as plsc

B = 16384
L = 20
NC = 2
NS = 16
NW = NC * NS
BPW = B // NW
CH = 128

ID_CHUNKS = BPW // CH
SMALL_ROWS = 5 * B
SMALL_IDX_PW = SMALL_ROWS // NW // CH
LAB_ROWS = 2 * B * L
LAB_IDX_PW = LAB_ROWS // NW // CH
LAB_INNER = 16
LAB_OUTER = LAB_IDX_PW // LAB_INNER
SMALL_INNER = 10
SMALL_OUTER = SMALL_IDX_PW // SMALL_INNER


def _mesh():
  return plsc.VectorSubcoreMesh(core_axis_name="c", subcore_axis_name="s",
                                num_cores=NC, num_subcores=NS)


def _wid():
  return lax.axis_index("s") * NC + lax.axis_index("c")


def _sc_gather_body(uid_idx, iid_idx, cat_idx, small_idx, lab_idx,
                    uid_tbl, iid_tbl, cat_tbl, small_tbl, lab_tbl,
                    uid_out, iid_out, cat_out, small_out, lab_out,
                    idxv, rows64, rows32, rowsc, rows16, sem):
  wid = _wid()

  def rows_gather(idx_hbm, tbl, out_hbm, rowsv):
    pltpu.sync_copy(idx_hbm.at[wid], idxv.at[pl.ds(0, ID_CHUNKS)])
    descs = []
    for j in range(ID_CHUNKS):
      descs.append(
          pltpu.async_copy(tbl.at[idxv.at[j]],
                           rowsv.at[pl.ds(j * CH, CH)], sem))
    for d in descs:
      d.wait()
    pltpu.sync_copy(rowsv, out_hbm.at[pl.ds(wid * ID_CHUNKS * CH,
                                            ID_CHUNKS * CH)])

  rows_gather(uid_idx, uid_tbl, uid_out, rows64)
  rows_gather(iid_idx, iid_tbl, iid_out, rows64)
  rows_gather(cat_idx, cat_tbl, cat_out, rowsc)

  pltpu.sync_copy(small_idx.at[wid], idxv.at[pl.ds(0, SMALL_IDX_PW)])

  def small_chunk(c, carry):
    descs = []
    for j in range(SMALL_INNER):
      descs.append(
          pltpu.async_copy(small_tbl.at[idxv.at[c * SMALL_INNER + j]],
                           rows16.at[pl.ds(j * CH, CH)], sem))
    for d in descs:
      d.wait()
    pltpu.sync_copy(
        rows16,
        small_out.at[pl.ds(wid * SMALL_IDX_PW * CH + c * SMALL_INNER * CH,
                           SMALL_INNER * CH)])
    return carry

  lax.fori_loop(0, SMALL_OUTER, small_chunk, 0)

  def lab_chunk(c, carry):
    pltpu.sync_copy(lab_idx.at[wid, pl.ds(c * LAB_INNER, LAB_INNER)],
                    idxv.at[pl.ds(0, LAB_INNER)])
    descs = []
    for j in range(LAB_INNER):
      descs.append(
          pltpu.async_copy(lab_tbl.at[idxv.at[j]],
                           rows32.at[pl.ds(j * CH, CH)], sem))
    for d in descs:
      d.wait()
    pltpu.sync_copy(
        rows32,
        lab_out.at[pl.ds(wid * LAB_IDX_PW * CH + c * LAB_INNER * CH,
                         LAB_INNER * CH)])
    return carry

  lax.fori_loop(0, LAB_OUTER, lab_chunk, 0)


def _sc_gather(uid_idx, iid_idx, cat_idx, small_idx, lab_idx,
               uid_tbl, iid_tbl, cat_tbl, small_tbl, lab_tbl):
  f = pl.kernel(
      _sc_gather_body,
      out_type=(
          jax.ShapeDtypeStruct((B, 64), jnp.float32),
          jax.ShapeDtypeStruct((B, 64), jnp.float32),
          jax.ShapeDtypeStruct((B, 32), jnp.float32),
          jax.ShapeDtypeStruct((SMALL_ROWS, 16), jnp.float32),
          jax.ShapeDtypeStruct((LAB_ROWS, 32), jnp.bfloat16),
      ),
      mesh=_mesh(),
      compiler_params=pltpu.CompilerParams(use_tc_tiling_on_sc=False),
      scratch_types=[
          pltpu.VMEM((SMALL_IDX_PW, CH), jnp.int32),
          pltpu.VMEM((ID_CHUNKS * CH, 64), jnp.float32),
          pltpu.VMEM((LAB_INNER * CH, 32), jnp.bfloat16),
          pltpu.VMEM((ID_CHUNKS * CH, 32), jnp.float32),
          pltpu.VMEM((SMALL_INNER * CH, 16), jnp.float32),
          pltpu.SemaphoreType.DMA,
      ],
  )
  return f(uid_idx, iid_idx, cat_idx, small_idx, lab_idx,
           uid_tbl, iid_tbl, cat_tbl, small_tbl, lab_tbl)


BS = 512


def _tc_dense_body(uid_ref, iid_ref, cat_ref, small_ref,
                   lab_ref, w20_ref, e20_ref, p32_ref,
                   u1_ref, ub1_ref, u2_ref, ub2_ref,
                   i1_ref, ib1_ref, i2_ref, ib2_ref, out_ref):
  uid_emb = uid_ref[...]
  iid_emb = iid_ref[...]

  w20 = w20_ref[...]
  e20 = e20_ref[...]
  p32 = p32_ref[...]

  def pool(x16):
    x = x16.astype(jnp.float32)
    s = jnp.dot(x, w20, preferred_element_type=jnp.float32)
    m = jnp.max(s, axis=1, keepdims=True)
    e = jnp.exp(s - m)
    wt = e / jnp.sum(e, axis=1, keepdims=True)
    wt640 = jnp.dot(wt, e20, preferred_element_type=jnp.float32)
    return jnp.dot(x * wt640, p32, preferred_element_type=jnp.float32)

  u_pool = pool(lab_ref[0])
  i_pool = pool(lab_ref[1])

  sm = small_ref[...]
  user_feat = jnp.concatenate(
      [uid_emb, sm[:, :64], u_pool], axis=1)
  item_feat = jnp.concatenate(
      [iid_emb, cat_ref[...], sm[:, 64:], i_pool], axis=1)

  hu = jnp.maximum(
      jnp.dot(user_feat, u1_ref[...], preferred_element_type=jnp.float32)
      + ub1_ref[0], 0.0)
  uvec = jnp.dot(hu, u2_ref[...], preferred_element_type=jnp.float32) \
      + ub2_ref[0]
  hi = jnp.dot(item_feat, i1_ref[...], preferred_element_type=jnp.float32) \
      + ib1_ref[0]
  ivec = jnp.dot(hi, i2_ref[...], preferred_element_type=jnp.float32) \
      + ib2_ref[0]
  logit = jnp.sum(uvec * ivec, axis=1, keepdims=True)
  out_ref[...] = 1.0 / (1.0 + jnp.exp(-logit))


def _tc_dense(uid_emb, iid_emb, cat_emb, small_emb, lab_emb, w20, e20, p32,
              U1, Ub1, U2, Ub2, I1, Ib1, I2, Ib2):
  grid = (B // BS,)
  full = lambda shape: pl.BlockSpec(shape, lambda i: tuple(0 for _ in shape))
  out = pl.pallas_call(
      _tc_dense_body,
      grid=grid,
      in_specs=[
          pl.BlockSpec((BS, 64), lambda i: (i, 0)),
          pl.BlockSpec((BS, 64), lambda i: (i, 0)),
          pl.BlockSpec((BS, 32), lambda i: (i, 0)),
          pl.BlockSpec((BS, 80), lambda i: (i, 0)),
          pl.BlockSpec((2, BS, 640), lambda i: (0, i, 0)),
          full((640, 20)), full((20, 640)), full((640, 32)),
          full((160, 256)), full((1, 256)), full((256, 128)), full((1, 128)),
          full((144, 256)), full((1, 256)), full((256, 128)), full((1, 128)),
      ],
      out_specs=pl.BlockSpec((BS, 1), lambda i: (i, 0)),
      out_shape=jax.ShapeDtypeStruct((B, 1), jnp.float32),
  )(uid_emb, iid_emb, cat_emb, small_emb, lab_emb, w20, e20, p32,
    U1, Ub1, U2, Ub2, I1, Ib1, I2, Ib2)
  return out


def kernel(user_id, gender_id, job_id, user_city_id, age_bucket, user_labels,
           item_id, category_id, item_city_id, item_labels,
           user_id_table, gender_table, job_table, city_table, age_table,
           item_id_table, category_table, label_table, w_pool,
           U1, Ub1, U2, Ub2, I1, Ib1, I2, Ib2):
  i32 = jnp.int32
  bf16 = jnp.bfloat16
  small_tbl = jnp.concatenate(
      [gender_table, job_table, city_table, age_table], axis=0)
  small_idx = jnp.stack([
      gender_id.astype(i32),
      job_id.astype(i32) + 3,
      user_city_id.astype(i32) + 104,
      age_bucket.astype(i32) + 1105,
      item_city_id.astype(i32) + 104,
  ], axis=1).reshape(NW, SMALL_IDX_PW, CH)
  lab_idx = jnp.concatenate(
      [user_labels.reshape(-1).astype(i32),
       item_labels.reshape(-1).astype(i32)]).reshape(NW, LAB_IDX_PW, CH)

  uid_emb, iid_emb, cat_emb, small_emb, lab_emb = _sc_gather(
      user_id.astype(i32).reshape(NW, ID_CHUNKS, CH),
      item_id.astype(i32).reshape(NW, ID_CHUNKS, CH),
      category_id.astype(i32).reshape(NW, ID_CHUNKS, CH),
      small_idx, lab_idx,
      user_id_table, item_id_table, category_table, small_tbl,
      label_table.astype(bf16))

  eye20 = jnp.eye(20, dtype=jnp.float32)
  w20 = jnp.kron(eye20, w_pool.reshape(32, 1))
  e20 = jnp.kron(eye20, jnp.ones((1, 32), jnp.float32))
  p32 = jnp.kron(jnp.ones((20, 1), jnp.float32),
                 jnp.eye(32, dtype=jnp.float32))

  out = _tc_dense(uid_emb, iid_emb, cat_emb,
                  small_emb.reshape(B, 80),
                  lab_emb.reshape(2, B, 640),
                  w20, e20, p32,
                  U1, Ub1.reshape(1, 256), U2, Ub2.reshape(1, 128),
                  I1, Ib1.reshape(1, 256), I2, Ib2.reshape(1, 128))
  return out.reshape(B)


# speedup vs baseline: 7.1318x; 1.0510x over previous
import functools

import jax
import jax.numpy as jnp
from jax import lax
from jax.experimental import pallas as pl
from jax.experimental.pallas import tpu as pltpu
from jax.experimental.pallas import tpu_sc as plsc

B = 16384
L = 20
NC = 2
NS = 16
NW = NC * NS
BPW = B // NW
CH = 128

ID_CHUNKS = BPW // CH
SMALL_ROWS = 5 * B
SMALL_IDX_PW = SMALL_ROWS // NW // CH
LAB_ROWS = 2 * B * L
LAB_IDX_PW = LAB_ROWS // NW // CH
LAB_INNER = 16
LAB_OUTER = LAB_IDX_PW // LAB_INNER
SMALL_INNER = 10
SMALL_OUTER = SMALL_IDX_PW // SMALL_INNER


def _mesh():
  return plsc.VectorSubcoreMesh(core_axis_name="c", subcore_axis_name="s",
                                num_cores=NC, num_subcores=NS)


def _wid():
  return lax.axis_index("s") * NC + lax.axis_index("c")


def _rows_gather(wid, idxv, idx_hbm, tbl, out_hbm, rowsv, sem):
  pltpu.sync_copy(idx_hbm.at[wid], idxv.at[pl.ds(0, ID_CHUNKS)])
  descs = []
  for j in range(ID_CHUNKS):
    descs.append(
        pltpu.async_copy(tbl.at[idxv.at[j]],
                         rowsv.at[pl.ds(j * CH, CH)], sem))
  for d in descs:
    d.wait()
  pltpu.sync_copy(rowsv, out_hbm.at[pl.ds(wid * ID_CHUNKS * CH,
                                          ID_CHUNKS * CH)])


def _sc_ids_body(uid_idx, iid_idx, uid_tbl, iid_tbl, uid_out, iid_out,
                 idxv, rows64, sem):
  wid = _wid()
  _rows_gather(wid, idxv, uid_idx, uid_tbl, uid_out, rows64, sem)
  _rows_gather(wid, idxv, iid_idx, iid_tbl, iid_out, rows64, sem)


def _sc_ids(uid_idx, iid_idx, uid_tbl, iid_tbl):
  f = pl.kernel(
      _sc_ids_body,
      out_type=(
          jax.ShapeDtypeStruct((B, 64), jnp.float32),
          jax.ShapeDtypeStruct((B, 64), jnp.float32),
      ),
      mesh=_mesh(),
      compiler_params=pltpu.CompilerParams(use_tc_tiling_on_sc=False),
      scratch_types=[
          pltpu.VMEM((ID_CHUNKS, CH), jnp.int32),
          pltpu.VMEM((ID_CHUNKS * CH, 64), jnp.float32),
          pltpu.SemaphoreType.DMA,
      ],
  )
  return f(uid_idx, iid_idx, uid_tbl, iid_tbl)


def _sc_gather_body(cat_idx, small_idx, lab_idx,
                    cat_tbl, small_tbl, lab_tbl,
                    cat_out, small_out, lab_out,
                    idxv, rows32, rowsc, rows16, sem):
  wid = _wid()

  _rows_gather(wid, idxv, cat_idx, cat_tbl, cat_out, rowsc, sem)

  pltpu.sync_copy(small_idx.at[wid], idxv.at[pl.ds(0, SMALL_IDX_PW)])

  def small_chunk(c, carry):
    descs = []
    for j in range(SMALL_INNER):
      descs.append(
          pltpu.async_copy(small_tbl.at[idxv.at[c * SMALL_INNER + j]],
                           rows16.at[pl.ds(j * CH, CH)], sem))
    for d in descs:
      d.wait()
    pltpu.sync_copy(
        rows16,
        small_out.at[pl.ds(wid * SMALL_IDX_PW * CH + c * SMALL_INNER * CH,
                           SMALL_INNER * CH)])
    return carry

  lax.fori_loop(0, SMALL_OUTER, small_chunk, 0)

  def lab_chunk(c, carry):
    pltpu.sync_copy(lab_idx.at[wid, pl.ds(c * LAB_INNER, LAB_INNER)],
                    idxv.at[pl.ds(0, LAB_INNER)])
    descs = []
    for j in range(LAB_INNER):
      descs.append(
          pltpu.async_copy(lab_tbl.at[idxv.at[j]],
                           rows32.at[pl.ds(j * CH, CH)], sem))
    for d in descs:
      d.wait()
    pltpu.sync_copy(
        rows32,
        lab_out.at[pl.ds(wid * LAB_IDX_PW * CH + c * LAB_INNER * CH,
                         LAB_INNER * CH)])
    return carry

  lax.fori_loop(0, LAB_OUTER, lab_chunk, 0)


def _sc_gather(cat_idx, small_idx, lab_idx, cat_tbl, small_tbl, lab_tbl):
  f = pl.kernel(
      _sc_gather_body,
      out_type=(
          jax.ShapeDtypeStruct((B, 32), jnp.float32),
          jax.ShapeDtypeStruct((SMALL_ROWS, 16), jnp.float32),
          jax.ShapeDtypeStruct((LAB_ROWS, 32), jnp.bfloat16),
      ),
      mesh=_mesh(),
      compiler_params=pltpu.CompilerParams(use_tc_tiling_on_sc=False),
      scratch_types=[
          pltpu.VMEM((SMALL_IDX_PW, CH), jnp.int32),
          pltpu.VMEM((LAB_INNER * CH, 32), jnp.bfloat16),
          pltpu.VMEM((ID_CHUNKS * CH, 32), jnp.float32),
          pltpu.VMEM((SMALL_INNER * CH, 16), jnp.float32),
          pltpu.SemaphoreType.DMA,
      ],
  )
  return f(cat_idx, small_idx, lab_idx, cat_tbl, small_tbl, lab_tbl)


BS = 512


def _tc_dense_body(uid_ref, iid_ref, cat_ref, small_ref,
                   lab_ref, w20_ref, e20_ref, p32_ref,
                   u1_ref, ub1_ref, u2_ref, ub2_ref,
                   i1_ref, ib1_ref, i2_ref, ib2_ref, out_ref):
  uid_emb = uid_ref[...]
  iid_emb = iid_ref[...]

  w20 = w20_ref[...]
  e20 = e20_ref[...]
  p32 = p32_ref[...]

  def pool(x16):
    x = x16.astype(jnp.float32)
    s = jnp.dot(x, w20, preferred_element_type=jnp.float32)
    m = jnp.max(s, axis=1, keepdims=True)
    e = jnp.exp(s - m)
    wt = e / jnp.sum(e, axis=1, keepdims=True)
    wt640 = jnp.dot(wt, e20, preferred_element_type=jnp.float32)
    return jnp.dot(x * wt640, p32, preferred_element_type=jnp.float32)

  u_pool = pool(lab_ref[0])
  i_pool = pool(lab_ref[1])

  sm = small_ref[...]
  user_feat = jnp.concatenate(
      [uid_emb, sm[:, :64], u_pool], axis=1)
  item_feat = jnp.concatenate(
      [iid_emb, cat_ref[...], sm[:, 64:], i_pool], axis=1)

  hu = jnp.maximum(
      jnp.dot(user_feat, u1_ref[...], preferred_element_type=jnp.float32)
      + ub1_ref[0], 0.0)
  uvec = jnp.dot(hu, u2_ref[...], preferred_element_type=jnp.float32) \
      + ub2_ref[0]
  hi = jnp.dot(item_feat, i1_ref[...], preferred_element_type=jnp.float32) \
      + ib1_ref[0]
  ivec = jnp.dot(hi, i2_ref[...], preferred_element_type=jnp.float32) \
      + ib2_ref[0]
  logit = jnp.sum(uvec * ivec, axis=1, keepdims=True)
  out_ref[...] = 1.0 / (1.0 + jnp.exp(-logit))


def _tc_dense(uid_emb, iid_emb, cat_emb, small_emb, lab_emb, w20, e20, p32,
              U1, Ub1, U2, Ub2, I1, Ib1, I2, Ib2):
  grid = (B // BS,)
  full = lambda shape: pl.BlockSpec(shape, lambda i: tuple(0 for _ in shape))
  out = pl.pallas_call(
      _tc_dense_body,
      grid=grid,
      in_specs=[
          pl.BlockSpec((BS, 64), lambda i: (i, 0)),
          pl.BlockSpec((BS, 64), lambda i: (i, 0)),
          pl.BlockSpec((BS, 32), lambda i: (i, 0)),
          pl.BlockSpec((BS, 80), lambda i: (i, 0)),
          pl.BlockSpec((2, BS, 640), lambda i: (0, i, 0)),
          full((640, 20)), full((20, 640)), full((640, 32)),
          full((160, 256)), full((1, 256)), full((256, 128)), full((1, 128)),
          full((144, 256)), full((1, 256)), full((256, 128)), full((1, 128)),
      ],
      out_specs=pl.BlockSpec((BS, 1), lambda i: (i, 0)),
      out_shape=jax.ShapeDtypeStruct((B, 1), jnp.float32),
  )(uid_emb, iid_emb, cat_emb, small_emb, lab_emb, w20, e20, p32,
    U1, Ub1, U2, Ub2, I1, Ib1, I2, Ib2)
  return out


def kernel(user_id, gender_id, job_id, user_city_id, age_bucket, user_labels,
           item_id, category_id, item_city_id, item_labels,
           user_id_table, gender_table, job_table, city_table, age_table,
           item_id_table, category_table, label_table, w_pool,
           U1, Ub1, U2, Ub2, I1, Ib1, I2, Ib2):
  i32 = jnp.int32
  bf16 = jnp.bfloat16
  small_tbl = jnp.concatenate(
      [gender_table, job_table, city_table, age_table], axis=0)
  small_idx = jnp.stack([
      gender_id.astype(i32),
      job_id.astype(i32) + 3,
      user_city_id.astype(i32) + 104,
      age_bucket.astype(i32) + 1105,
      item_city_id.astype(i32) + 104,
  ], axis=1).reshape(NW, SMALL_IDX_PW, CH)
  lab_idx = jnp.concatenate(
      [user_labels.reshape(-1).astype(i32),
       item_labels.reshape(-1).astype(i32)]).reshape(NW, LAB_IDX_PW, CH)

  cat_emb, small_emb, lab_emb = _sc_gather(
      category_id.astype(i32).reshape(NW, ID_CHUNKS, CH),
      small_idx, lab_idx,
      category_table, small_tbl, label_table.astype(bf16))
  uid_emb, iid_emb = _sc_ids(
      user_id.astype(i32).reshape(NW, ID_CHUNKS, CH),
      item_id.astype(i32).reshape(NW, ID_CHUNKS, CH),
      user_id_table, item_id_table)

  eye20 = jnp.eye(20, dtype=jnp.float32)
  w20 = jnp.kron(eye20, w_pool.reshape(32, 1))
  e20 = jnp.kron(eye20, jnp.ones((1, 32), jnp.float32))
  p32 = jnp.kron(jnp.ones((20, 1), jnp.float32),
                 jnp.eye(32, dtype=jnp.float32))

  out = _tc_dense(uid_emb, iid_emb, cat_emb,
                  small_emb.reshape(B, 80),
                  lab_emb.reshape(2, B, 640),
                  w20, e20, p32,
                  U1, Ub1.reshape(1, 256), U2, Ub2.reshape(1, 128),
                  I1, Ib1.reshape(1, 256), I2, Ib2.reshape(1, 128))
  return out.reshape(B)
